# Initial kernel scaffold; baseline (speedup 1.0000x reference)
import jax
import jax.numpy as jnp
from jax.experimental import pallas as pl


def kernel(x, W1, b1, W2, batch):
    raise NotImplementedError("write your pallas kernel here")


# speedup vs baseline: 14.2324x; 14.2324x over previous
import jax
import jax.numpy as jnp
from jax import lax
from jax.experimental import pallas as pl
from jax.experimental.pallas import tpu as pltpu

_N = 100000
_D = 128
_S = 256
_B = 2000


def _tc_kernel(x_ref, w1_ref, b1_ref, w2_ref, batch_ref, out_ref, acc_ref, den_ref):
    i = pl.program_id(0)
    nb = pl.num_programs(0)

    @pl.when(i == 0)
    def _init():
        acc_ref[...] = jnp.zeros_like(acc_ref)
        den_ref[...] = jnp.zeros_like(den_ref)

    x = x_ref[...]
    w2 = w2_ref[...]
    h = jnp.tanh(
        jnp.dot(x, w1_ref[...], preferred_element_type=jnp.float32) + b1_ref[...]
    )
    s = jnp.dot(h, w2, preferred_element_type=jnp.float32)
    c = jnp.sum(jnp.abs(w2))
    e = jnp.exp(s - c)

    seg = batch_ref[...]
    cols = lax.broadcasted_iota(jnp.int32, (_B, _S), 1)
    P = jnp.where(seg == cols, e, 0.0)

    acc_ref[...] += lax.dot_general(
        P, x, (((0,), (0,)), ((), ())), preferred_element_type=jnp.float32
    )
    den_ref[...] += lax.dot_general(
        P, jnp.ones((_B, 8), jnp.float32), (((0,), (0,)), ((), ())),
        preferred_element_type=jnp.float32,
    )

    @pl.when(i == nb - 1)
    def _fin():
        out_ref[...] = acc_ref[...] / (den_ref[:, 0:1] + 1e-16)


def kernel(x, W1, b1, W2, batch):
    batch2 = batch.astype(jnp.int32).reshape(_N, 1)
    b1r = b1.reshape(1, _D)
    nb = _N // _B
    return pl.pallas_call(
        _tc_kernel,
        grid=(nb,),
        in_specs=[
            pl.BlockSpec((_B, _D), lambda i: (i, 0)),
            pl.BlockSpec((_D, _D), lambda i: (0, 0)),
            pl.BlockSpec((1, _D), lambda i: (0, 0)),
            pl.BlockSpec((_D, 1), lambda i: (0, 0)),
            pl.BlockSpec((_B, 1), lambda i: (i, 0)),
        ],
        out_specs=pl.BlockSpec((_S, _D), lambda i: (0, 0)),
        out_shape=jax.ShapeDtypeStruct((_S, _D), jnp.float32),
        scratch_shapes=[
            pltpu.VMEM((_S, _D), jnp.float32),
            pltpu.VMEM((_S, 8), jnp.float32),
        ],
        compiler_params=pltpu.CompilerParams(
            dimension_semantics=("arbitrary",),
        ),
    )(x, W1, b1r, W2, batch2)


# speedup vs baseline: 27.8945x; 1.9599x over previous
import jax
import jax.numpy as jnp
from jax import lax
from jax.experimental import pallas as pl
from jax.experimental.pallas import tpu as pltpu

_N = 100000
_D = 128
_S = 256
_B = 2000


def _tc_kernel(x_ref, w1_ref, b1_ref, w2_ref, batch_ref, out_ref, acc_ref, den_ref):
    i = pl.program_id(0)
    nb = pl.num_programs(0)

    @pl.when(i == 0)
    def _init():
        acc_ref[...] = jnp.zeros_like(acc_ref)
        den_ref[...] = jnp.zeros_like(den_ref)

    x = x_ref[...]
    w2t = w2_ref[...]
    h = jnp.tanh(
        jnp.dot(x, w1_ref[...], preferred_element_type=jnp.float32) + b1_ref[...]
    )
    st = lax.dot_general(
        w2t, h, (((1,), (1,)), ((), ())), preferred_element_type=jnp.float32
    )
    c = jnp.sum(jnp.abs(w2t))
    e = jnp.exp(st - c)

    seg = batch_ref[...].reshape(1, _B)
    rows = lax.broadcasted_iota(jnp.int32, (_S, _B), 0)
    Pt = jnp.where(seg == rows, e, 0.0)

    acc_ref[...] += lax.dot_general(
        Pt, x, (((1,), (0,)), ((), ())), preferred_element_type=jnp.float32
    )
    den_ref[:, 0:1] += jnp.sum(Pt, axis=1, keepdims=True)

    @pl.when(i == nb - 1)
    def _fin():
        out_ref[...] = acc_ref[...] / (den_ref[:, 0:1] + 1e-16)


def kernel(x, W1, b1, W2, batch):
    nb = _N // _B
    batch2 = batch.astype(jnp.int32).reshape(nb, 1, _B)
    b1r = b1.reshape(1, _D)
    w2t = W2.reshape(1, _D)
    return pl.pallas_call(
        _tc_kernel,
        grid=(nb,),
        in_specs=[
            pl.BlockSpec((_B, _D), lambda i: (i, 0)),
            pl.BlockSpec((_D, _D), lambda i: (0, 0)),
            pl.BlockSpec((1, _D), lambda i: (0, 0)),
            pl.BlockSpec((1, _D), lambda i: (0, 0)),
            pl.BlockSpec((1, 1, _B), lambda i: (i, 0, 0)),
        ],
        out_specs=pl.BlockSpec((_S, _D), lambda i: (0, 0)),
        out_shape=jax.ShapeDtypeStruct((_S, _D), jnp.float32),
        scratch_shapes=[
            pltpu.VMEM((_S, _D), jnp.float32),
            pltpu.VMEM((_S, 8), jnp.float32),
        ],
        compiler_params=pltpu.CompilerParams(
            dimension_semantics=("arbitrary",),
        ),
    )(x, W1, b1r, w2t, batch2)
